# baseline (speedup 1.0000x reference)
import functools

import jax
import jax.numpy as jnp
from jax import lax
from jax.experimental import pallas as pl
from jax.experimental.pallas import tpu as pltpu
from jax.experimental.pallas import tpu_sc as plsc

DT = 0.1
TAU_HP = 12.3
TAU_LP = 2.3

NC = 2
NS = 16
NW = NC * NS
LANES = 16
ROW = 128
ROWS = 8
E_CH = ROW * ROWS


def _round_up(x: int, m: int) -> int:
    return (x + m - 1) // m * m


@functools.lru_cache(maxsize=None)
def _build_step(n_neurons: int, acc_len: int, rows_per_tile: int, n_ch: int):
    grp = acc_len // NS
    mesh = plsc.VectorSubcoreMesh(
        core_axis_name="c", subcore_axis_name="s",
        num_cores=NC, num_subcores=NS,
    )

    @functools.partial(
        pl.kernel,
        out_type=jax.ShapeDtypeStruct((NC * acc_len,), jnp.float32),
        mesh=mesh,
        compiler_params=pltpu.CompilerParams(needs_layout_passes=False),
        scratch_types=[
            pltpu.VMEM((n_neurons,), jnp.float32),
            [pltpu.VMEM((2, ROWS, ROW), jnp.int32)] * 4,
            [pltpu.VMEM((E_CH // 2,), jnp.int32)] * 4,
            [pltpu.VMEM((ROWS, ROW), jnp.float32)] * 4,
            pltpu.VMEM((E_CH,), jnp.float32),
            pltpu.VMEM_SHARED((acc_len,), jnp.float32),
            pltpu.SemaphoreType.DMA,
            [pltpu.SemaphoreType.DMA] * 4,
            [pltpu.SemaphoreType.DMA] * 2,
        ],
    )
    def step(r_hbm, edges_hbm, w_hbm, out_hbm,
             r_v, e_v, w_v, val_v, z_v, acc,
             sem_r, sems_in, sems_sc):
        cid = lax.axis_index("c")
        sid = lax.axis_index("s")
        wid = cid * NS + sid
        c0 = wid * n_ch

        def in_descs(i, s):
            return (
                pltpu.make_async_copy(
                    edges_hbm.at[pl.ds((c0 + i) * 2, 2)], e_v[s], sems_in[s]),
                pltpu.make_async_copy(
                    w_hbm.at[pl.ds((c0 + i) * (E_CH // 2), E_CH // 2)],
                    w_v[s], sems_in[s]),
            )

        def scat_descs(s, p):
            return [
                pltpu.make_async_copy(
                    val_v[s].at[j], acc.at[e_v[s].at[1, j]], sems_sc[p])
                for j in range(ROWS)
            ]

        for i in range(2):
            for d in in_descs(i, i):
                d.start()
        pltpu.make_async_copy(r_hbm, r_v, sem_r).start()

        zero16 = jnp.zeros((LANES,), jnp.float32)

        def _zloop(k, c):
            z_v[pl.ds(k * LANES, LANES)] = zero16
            return c

        lax.fori_loop(0, E_CH // LANES, _zloop, 0)
        for off in range(0, grp, E_CH):
            n = min(E_CH, grp - off)
            pltpu.sync_copy(z_v.at[pl.ds(0, n)],
                            acc.at[pl.ds(sid * grp + off, n)])
        plsc.subcore_barrier()
        pltpu.make_async_copy(r_hbm, r_v, sem_r).wait()

        def body(g, c):
            for u in range(4):
                i = 4 * g + u
                p = u % 2
                for d in in_descs(i, u):
                    d.wait()
                @pl.when(i >= 2)
                def _():
                    for d in scat_descs((u + 2) % 4, p):
                        d.wait()
                @pl.when(i + 2 < n_ch)
                def _():
                    for d in in_descs(i + 2, (u + 2) % 4):
                        d.start()
                for j in range(ROWS):
                    for k in range(ROW // (2 * LANES)):
                        sla = pl.ds(k * 2 * LANES, LANES)
                        slb = pl.ds(k * 2 * LANES + LANES, LANES)
                        wp = w_v[u][pl.ds((j * ROW // 2) + k * LANES, LANES)]
                        wa = plsc.bitcast(wp << 16, jnp.float32)
                        wb = plsc.bitcast(wp & jnp.int32(-65536), jnp.float32)
                        val_v[u][j, sla] = (
                            plsc.load_gather(r_v, [e_v[u][0, j, sla]]) * wa)
                        val_v[u][j, slb] = (
                            plsc.load_gather(r_v, [e_v[u][0, j, slb]]) * wb)
                for d in scat_descs(u, p):
                    d.start(add=True)
            return c

        lax.fori_loop(0, n_ch // 4, body, 0)

        for i in (n_ch - 2, n_ch - 1):
            for d in scat_descs(i % 4, i % 2):
                d.wait()

        plsc.subcore_barrier()
        for off in range(0, grp, E_CH):
            n = min(E_CH, grp - off)
            pltpu.sync_copy(acc.at[pl.ds(sid * grp + off, n)],
                            z_v.at[pl.ds(0, n)])
            pltpu.sync_copy(z_v.at[pl.ds(0, n)],
                            out_hbm.at[pl.ds(cid * acc_len + sid * grp + off, n)])

    return step


def kernel(tm1_input, source_indices, target_indices, weights, tau, vrest,
           edge_scales):
    n_tm1 = tm1_input.shape[1]
    n_neurons = tau.shape[0]
    n_edges = source_indices.shape[0]

    e_w = _round_up(-(-n_edges // NW), E_CH)
    pad = e_w * NW - n_edges
    acc_len = _round_up(n_neurons + 1, NS * 8)
    step_pallas = _build_step(n_neurons, acc_len, e_w // ROW, e_w // E_CH)

    sw = weights * edge_scales
    n_chunks = (e_w * NW) // E_CH
    src = jnp.pad(source_indices, (0, pad)).reshape(n_chunks, 1, ROWS, ROW)
    tgt = jnp.pad(target_indices, (0, pad),
                  constant_values=n_neurons).reshape(n_chunks, 1, ROWS, ROW)
    edges = jnp.concatenate([src, tgt], axis=1).reshape(-1, ROWS, ROW)
    w_bf = jnp.pad(sw, (0, pad)).astype(jnp.bfloat16).reshape(-1, 2, LANES)
    w_pk = jax.lax.bitcast_convert_type(
        jnp.stack([w_bf[:, 0, :], w_bf[:, 1, :]], axis=-1), jnp.int32).ravel()

    def step(carry, x):
        v, f, tv = carry
        hp = x - f
        f = f + DT * hp / TAU_HP
        rect = jnp.maximum(hp, 0.0)
        v = jnp.concatenate([tv, v[n_tm1:]])
        tv_new = tv + DT * (rect - tv) / TAU_LP
        r = jnp.maximum(v, 0.0)
        part = step_pallas(r, edges, w_pk)
        summed = part[:n_neurons] + part[acc_len:acc_len + n_neurons]
        v = v + DT * (vrest - v + summed) / tau
        v = jnp.concatenate([tv_new, v[n_tm1:]])
        return (v, f, tv_new), None

    v0 = jnp.zeros((n_neurons,), jnp.float32)
    f0 = jnp.zeros((n_tm1,), jnp.float32)
    tv0 = jnp.zeros((n_tm1,), jnp.float32)
    (v, _, _), _ = lax.scan(step, (v0, f0, tv0), tm1_input)
    return v[None, :]


# speedup vs baseline: 1.4994x; 1.4994x over previous
import functools

import jax
import jax.numpy as jnp
from jax import lax
from jax.experimental import pallas as pl
from jax.experimental.pallas import tpu as pltpu
from jax.experimental.pallas import tpu_sc as plsc

DT = 0.1
TAU_HP = 12.3
TAU_LP = 2.3

NC = 2
NS = 16
NW = NC * NS
LANES = 16
ROW = 128
ROWS = 8
E_CH = ROW * ROWS


def _round_up(x: int, m: int) -> int:
    return (x + m - 1) // m * m


@functools.lru_cache(maxsize=None)
def _build_step(n_neurons: int, acc_len: int, rows_per_tile: int, n_ch: int):
    grp = acc_len // NS
    mesh = plsc.VectorSubcoreMesh(
        core_axis_name="c", subcore_axis_name="s",
        num_cores=NC, num_subcores=NS,
    )

    @functools.partial(
        pl.kernel,
        out_type=jax.ShapeDtypeStruct((NC * acc_len,), jnp.float32),
        mesh=mesh,
        compiler_params=pltpu.CompilerParams(needs_layout_passes=False),
        scratch_types=[
            pltpu.VMEM((n_neurons,), jnp.float32),
            [pltpu.VMEM((3, ROWS, ROW), jnp.int32)] * 4,
            [pltpu.VMEM((ROWS, ROW), jnp.float32)] * 4,
            pltpu.VMEM((E_CH,), jnp.float32),
            pltpu.VMEM_SHARED((acc_len,), jnp.float32),
            pltpu.SemaphoreType.DMA,
            [pltpu.SemaphoreType.DMA] * 4,
            [pltpu.SemaphoreType.DMA] * 2,
        ],
    )
    def step(r_hbm, edges_hbm, out_hbm,
             r_v, e_v, val_v, z_v, acc,
             sem_r, sems_in, sems_sc):
        cid = lax.axis_index("c")
        sid = lax.axis_index("s")
        wid = cid * NS + sid
        c0 = wid * n_ch

        def in_desc(i, s):
            return pltpu.make_async_copy(
                edges_hbm.at[pl.ds((c0 + i) * 3, 3)], e_v[s], sems_in[s])

        def scat_descs(s, p):
            return [
                pltpu.make_async_copy(
                    val_v[s].at[j], acc.at[e_v[s].at[1, j]], sems_sc[p])
                for j in range(ROWS)
            ]

        for i in range(2):
            in_desc(i, i).start()
        pltpu.make_async_copy(r_hbm, r_v, sem_r).start()

        zero16 = jnp.zeros((LANES,), jnp.float32)

        def _zloop(k, c):
            z_v[pl.ds(k * LANES, LANES)] = zero16
            return c

        lax.fori_loop(0, E_CH // LANES, _zloop, 0)
        for off in range(0, grp, E_CH):
            n = min(E_CH, grp - off)
            pltpu.sync_copy(z_v.at[pl.ds(0, n)],
                            acc.at[pl.ds(sid * grp + off, n)])
        plsc.subcore_barrier()
        pltpu.make_async_copy(r_hbm, r_v, sem_r).wait()

        def body(g, c):
            for u in range(4):
                i = 4 * g + u
                p = u % 2
                in_desc(i, u).wait()
                @pl.when(i >= 2)
                def _():
                    for d in scat_descs((u + 2) % 4, p):
                        d.wait()
                @pl.when(i + 2 < n_ch)
                def _():
                    in_desc(i + 2, (u + 2) % 4).start()
                for j in range(ROWS):
                    for k in range(ROW // LANES):
                        sl = pl.ds(k * LANES, LANES)
                        idx = e_v[u][0, j, sl]
                        wv = plsc.bitcast(e_v[u][2, j, sl], jnp.float32)
                        val_v[u][j, sl] = plsc.load_gather(r_v, [idx]) * wv
                for d in scat_descs(u, p):
                    d.start(add=True)
            return c

        lax.fori_loop(0, n_ch // 4, body, 0)

        for i in (n_ch - 2, n_ch - 1):
            for d in scat_descs(i % 4, i % 2):
                d.wait()

        plsc.subcore_barrier()
        for off in range(0, grp, E_CH):
            n = min(E_CH, grp - off)
            pltpu.sync_copy(acc.at[pl.ds(sid * grp + off, n)],
                            z_v.at[pl.ds(0, n)])
            pltpu.sync_copy(z_v.at[pl.ds(0, n)],
                            out_hbm.at[pl.ds(cid * acc_len + sid * grp + off, n)])

    return step


def kernel(tm1_input, source_indices, target_indices, weights, tau, vrest,
           edge_scales):
    n_tm1 = tm1_input.shape[1]
    n_neurons = tau.shape[0]
    n_edges = source_indices.shape[0]

    e_w = _round_up(-(-n_edges // NW), E_CH)
    pad = e_w * NW - n_edges
    acc_len = _round_up(n_neurons + 1, NS * 8)
    step_pallas = _build_step(n_neurons, acc_len, e_w // ROW, e_w // E_CH)

    sw = weights * edge_scales
    n_chunks = (e_w * NW) // E_CH
    src = jnp.pad(source_indices, (0, pad)).reshape(n_chunks, 1, ROWS, ROW)
    tgt = jnp.pad(target_indices, (0, pad),
                  constant_values=n_neurons).reshape(n_chunks, 1, ROWS, ROW)
    wi = jax.lax.bitcast_convert_type(jnp.pad(sw, (0, pad)),
                                      jnp.int32).reshape(n_chunks, 1, ROWS, ROW)
    edges = jnp.concatenate([src, tgt, wi], axis=1).reshape(-1, ROWS, ROW)

    def step(carry, x):
        v, f, tv = carry
        hp = x - f
        f = f + DT * hp / TAU_HP
        rect = jnp.maximum(hp, 0.0)
        v = jnp.concatenate([tv, v[n_tm1:]])
        tv_new = tv + DT * (rect - tv) / TAU_LP
        r = jnp.maximum(v, 0.0)
        part = step_pallas(r, edges)
        summed = part[:n_neurons] + part[acc_len:acc_len + n_neurons]
        v = v + DT * (vrest - v + summed) / tau
        v = jnp.concatenate([tv_new, v[n_tm1:]])
        return (v, f, tv_new), None

    v0 = jnp.zeros((n_neurons,), jnp.float32)
    f0 = jnp.zeros((n_tm1,), jnp.float32)
    tv0 = jnp.zeros((n_tm1,), jnp.float32)
    (v, _, _), _ = lax.scan(step, (v0, f0, tv0), tm1_input)
    return v[None, :]
